# baseline (speedup 1.0000x reference)
import functools

import jax
import jax.numpy as jnp
from jax import lax
from jax.experimental import pallas as pl
from jax.experimental.pallas import tpu as pltpu
from jax.experimental.pallas import tpu_sc as plsc

N_CODE = 1024
D_K = 256


_RB = 1024


def _dist_body(v_ref, c_ref, cn_ref, z_ref, lsum_ref):
    i = pl.program_id(0)

    @pl.when(i == 0)
    def _init():
        lsum_ref[0, 0] = 0.0

    v = v_ref[...]
    c = c_ref[...]
    s = lax.dot_general(v, c, (((1,), (1,)), ((), ())),
                        preferred_element_type=jnp.float32)
    vn = jnp.sum(v * v, axis=1, keepdims=True)
    diffs = (vn + (-2.0) * s) + cn_ref[...][None, :]
    m = jnp.min(diffs, axis=1, keepdims=True)
    ids = lax.broadcasted_iota(jnp.int32, (_RB, N_CODE), 1)
    z = jnp.min(jnp.where(diffs == m, ids, N_CODE), axis=1)
    z_ref[...] = z.astype(jnp.int32)
    lsum_ref[0, 0] += jnp.sum(jnp.maximum(m, 0.0))


def _distances_argmin(v2, c, cn):
    n = v2.shape[0]
    grid = (n // _RB,)
    z, lsum = pl.pallas_call(
        _dist_body,
        grid=grid,
        in_specs=[
            pl.BlockSpec((_RB, D_K), lambda i: (i, 0)),
            pl.BlockSpec((N_CODE, D_K), lambda i: (0, 0)),
            pl.BlockSpec((N_CODE,), lambda i: (0,)),
        ],
        out_specs=[
            pl.BlockSpec((_RB,), lambda i: (i,)),
            pl.BlockSpec(memory_space=pltpu.SMEM),
        ],
        out_shape=[
            jax.ShapeDtypeStruct((n,), jnp.int32),
            jax.ShapeDtypeStruct((1, 1), jnp.float32),
        ],
    )(v2, c, cn)
    return z, lsum


_NC = 2
_NS = 16
_NW = _NC * _NS
_CHUNK = 128


def _sc_gather(table, idx3, n_rows):
    b_per_w = n_rows // _NW
    n_chunk = b_per_w // _CHUNK
    mesh = plsc.VectorSubcoreMesh(core_axis_name="c", subcore_axis_name="s")

    @functools.partial(
        pl.kernel,
        mesh=mesh,
        out_type=jax.ShapeDtypeStruct((n_rows, D_K), jnp.float32),
        scratch_types=[
            pltpu.VMEM((n_chunk, _CHUNK), jnp.int32),
            pltpu.VMEM((_CHUNK, D_K), jnp.float32),
            pltpu.VMEM((_CHUNK, D_K), jnp.float32),
            pltpu.SemaphoreType.DMA,
            pltpu.SemaphoreType.DMA,
        ],
    )
    def gather_k(table_hbm, idx_hbm, out_hbm, idx_v, buf0, buf1, sem0, sem1):
        wid = lax.axis_index("s") * _NC + lax.axis_index("c")
        base = wid * b_per_w
        pltpu.sync_copy(idx_hbm.at[wid], idx_v)
        bufs = (buf0, buf1)
        sems = (sem0, sem1)
        cp = pltpu.async_copy(table_hbm.at[idx_v.at[0]], bufs[0], sems[0])
        for j in range(n_chunk):
            cur = cp
            if j + 1 < n_chunk:
                cp = pltpu.async_copy(
                    table_hbm.at[idx_v.at[j + 1]], bufs[(j + 1) % 2],
                    sems[(j + 1) % 2])
            cur.wait()
            pltpu.sync_copy(bufs[j % 2],
                            out_hbm.at[pl.ds(base + j * _CHUNK, _CHUNK)])

    return gather_k(table, idx3)


_NSPLIT = 4


def kernel(vecs, c_sum, c_count):
    b, r, cdim, k = vecs.shape
    n = b * r * cdim
    v2 = vecs.astype(jnp.float32).reshape(n, k)
    c = jnp.divide(c_sum, jnp.clip(jnp.expand_dims(c_count, -1), 0.01))
    c = c.astype(jnp.float32)
    cn = jnp.einsum('sk->s', jnp.square(c))

    np_ = n // _NSPLIT
    z_parts, hat_parts, lsums = [], [], []
    for p in range(_NSPLIT):
        z_p, lsum_p = _distances_argmin(
            lax.slice_in_dim(v2, p * np_, (p + 1) * np_, axis=0), c, cn)
        idx3 = z_p.reshape(_NW, np_ // (_NW * _CHUNK), _CHUNK)
        hat_parts.append(_sc_gather(c, idx3, np_))
        z_parts.append(z_p)
        lsums.append(lsum_p[0, 0])

    l_commit = sum(lsums) / (b * r)
    z = jnp.concatenate(z_parts).reshape(b, r, cdim)
    vecs_hat = jnp.concatenate(hat_parts).reshape(b, r, cdim, k)
    vecs_hat = vecs_hat.astype(vecs.dtype)
    l_codebook = jnp.zeros((), jnp.float32)
    return vecs_hat, z, l_commit, l_codebook


# speedup vs baseline: 1.0905x; 1.0905x over previous
import functools

import jax
import jax.numpy as jnp
from jax import lax
from jax.experimental import pallas as pl
from jax.experimental.pallas import tpu as pltpu
from jax.experimental.pallas import tpu_sc as plsc

N_CODE = 1024
D_K = 256


_RB = 1024


def _dist_body(v_ref, c_ref, cn_ref, z_ref, lsum_ref):
    i = pl.program_id(0)

    @pl.when(i == 0)
    def _init():
        lsum_ref[0, 0] = 0.0

    v = v_ref[...]
    c = c_ref[...]
    s = lax.dot_general(v, c, (((1,), (1,)), ((), ())),
                        preferred_element_type=jnp.float32)
    vn = jnp.sum(v * v, axis=1, keepdims=True)
    diffs = (vn + (-2.0) * s) + cn_ref[...][None, :]
    m = jnp.min(diffs, axis=1, keepdims=True)
    ids = lax.broadcasted_iota(jnp.int32, (_RB, N_CODE), 1)
    z = jnp.min(jnp.where(diffs == m, ids, N_CODE), axis=1)
    z_ref[...] = z.astype(jnp.int32)
    lsum_ref[0, 0] += jnp.sum(jnp.maximum(m, 0.0))


def _distances_argmin(v2, c, cn, part, n_part):
    steps = n_part // _RB
    grid = (steps,)
    z, lsum = pl.pallas_call(
        _dist_body,
        grid=grid,
        in_specs=[
            pl.BlockSpec((_RB, D_K), lambda i, p=part, s=steps: (p * s + i, 0)),
            pl.BlockSpec((N_CODE, D_K), lambda i: (0, 0)),
            pl.BlockSpec((N_CODE,), lambda i: (0,)),
        ],
        out_specs=[
            pl.BlockSpec((_RB,), lambda i: (i,)),
            pl.BlockSpec(memory_space=pltpu.SMEM),
        ],
        out_shape=[
            jax.ShapeDtypeStruct((n_part,), jnp.int32),
            jax.ShapeDtypeStruct((1, 1), jnp.float32),
        ],
    )(v2, c, cn)
    return z, lsum


_NC = 2
_NS = 16
_NW = _NC * _NS
_CHUNK = 64


def _sc_gather(table, idx3, n_rows, out_rows, row0):
    b_per_w = n_rows // _NW
    n_chunk = b_per_w // _CHUNK
    mesh = plsc.VectorSubcoreMesh(core_axis_name="c", subcore_axis_name="s")

    @functools.partial(
        pl.kernel,
        mesh=mesh,
        out_type=jax.ShapeDtypeStruct((out_rows, D_K), jnp.float32),
        scratch_types=[
            pltpu.VMEM((n_chunk, _CHUNK), jnp.int32),
            pltpu.VMEM((_CHUNK, D_K), jnp.float32),
            pltpu.VMEM((_CHUNK, D_K), jnp.float32),
            pltpu.SemaphoreType.DMA,
            pltpu.SemaphoreType.DMA,
            pltpu.SemaphoreType.DMA,
            pltpu.SemaphoreType.DMA,
        ],
    )
    def gather_k(table_hbm, idx_hbm, out_hbm, idx_v, buf0, buf1,
                 gs0, gs1, ws0, ws1):
        wid = lax.axis_index("s") * _NC + lax.axis_index("c")
        base = row0 + wid * b_per_w
        pltpu.sync_copy(idx_hbm.at[wid], idx_v)
        bufs = (buf0, buf1)
        gsem = (gs0, gs1)
        wsem = (ws0, ws1)

        def start_write(j):
            return pltpu.async_copy(
                bufs[j % 2], out_hbm.at[pl.ds(base + j * _CHUNK, _CHUNK)],
                wsem[j % 2])

        g, w = {}, {}
        for j in range(n_chunk):
            if j >= 2:
                w[j - 2].wait()
            g[j] = pltpu.async_copy(
                table_hbm.at[idx_v.at[j]], bufs[j % 2], gsem[j % 2])
            if j >= 1:
                g[j - 1].wait()
                w[j - 1] = start_write(j - 1)
        g[n_chunk - 1].wait()
        w[n_chunk - 1] = start_write(n_chunk - 1)
        if n_chunk >= 2:
            w[n_chunk - 2].wait()
        w[n_chunk - 1].wait()

    return gather_k(table, idx3)


_NSPLIT = 4


def kernel(vecs, c_sum, c_count):
    b, r, cdim, k = vecs.shape
    n = b * r * cdim
    v2 = vecs.astype(jnp.float32).reshape(n, k)
    c = jnp.divide(c_sum, jnp.clip(jnp.expand_dims(c_count, -1), 0.01))
    c = c.astype(jnp.float32)
    cn = jnp.einsum('sk->s', jnp.square(c))

    np_ = n // _NSPLIT
    z_parts, lsums = [], []
    hat = None
    for p in range(_NSPLIT):
        z_p, lsum_p = _distances_argmin(v2, c, cn, p, np_)
        idx3 = z_p.reshape(_NW, np_ // (_NW * _CHUNK), _CHUNK)
        if p == 0:
            hat = _sc_gather(c, idx3, np_, n, 0)
        else:
            hat_p = _sc_gather(c, idx3, np_, np_, 0)
            hat = lax.dynamic_update_slice(hat, hat_p, (p * np_, 0))
        z_parts.append(z_p)
        lsums.append(lsum_p[0, 0])

    l_commit = sum(lsums) / (b * r)
    z = jnp.concatenate(z_parts).reshape(b, r, cdim)
    vecs_hat = hat.reshape(b, r, cdim, k).astype(vecs.dtype)
    l_codebook = jnp.zeros((), jnp.float32)
    return vecs_hat, z, l_commit, l_codebook


# speedup vs baseline: 1.1758x; 1.0781x over previous
import functools

import jax
import jax.numpy as jnp
from jax import lax
from jax.experimental import pallas as pl
from jax.experimental.pallas import tpu as pltpu
from jax.experimental.pallas import tpu_sc as plsc

N_CODE = 1024
D_K = 256


_RB = 1024
_NCH = 128
_NCHUNKS = N_CODE // _NCH


def _dist_body(v_ref, c_ref, cn_ref, z_ref, lsum_ref):
    i = pl.program_id(0)

    @pl.when(i == 0)
    def _init():
        lsum_ref[0, 0] = 0.0

    v = v_ref[...]
    c = c_ref[...]
    s = lax.dot_general(v, c, (((1,), (1,)), ((), ())),
                        preferred_element_type=jnp.float32)
    vn = jnp.sum(v * v, axis=1, keepdims=True)
    cn = cn_ref[...]

    def chunk(j):
        lo, hi = j * _NCH, (j + 1) * _NCH
        return (vn + (-2.0) * s[:, lo:hi]) + cn[lo:hi][None, :]

    val = chunk(0)
    jwin = jnp.zeros((_RB, _NCH), jnp.int32)
    for j in range(1, _NCHUNKS):
        d = chunk(j)
        lt = d < val
        val = jnp.minimum(val, d)
        jwin = jnp.where(lt, j, jwin)
    g = jwin * _NCH + lax.broadcasted_iota(jnp.int32, (_RB, _NCH), 1)
    m = jnp.min(val, axis=1, keepdims=True)
    z = jnp.min(jnp.where(val == m, g, N_CODE), axis=1)
    z_ref[...] = z.astype(jnp.int32)
    lsum_ref[0, 0] += jnp.sum(jnp.maximum(m, 0.0))


def _distances_argmin(v2, c, cn, part, n_part):
    steps = n_part // _RB
    grid = (steps,)
    z, lsum = pl.pallas_call(
        _dist_body,
        grid=grid,
        in_specs=[
            pl.BlockSpec((_RB, D_K), lambda i, p=part, s=steps: (p * s + i, 0)),
            pl.BlockSpec((N_CODE, D_K), lambda i: (0, 0)),
            pl.BlockSpec((N_CODE,), lambda i: (0,)),
        ],
        out_specs=[
            pl.BlockSpec((_RB,), lambda i: (i,)),
            pl.BlockSpec(memory_space=pltpu.SMEM),
        ],
        out_shape=[
            jax.ShapeDtypeStruct((n_part,), jnp.int32),
            jax.ShapeDtypeStruct((1, 1), jnp.float32),
        ],
    )(v2, c, cn)
    return z, lsum


_NC = 2
_NS = 16
_NW = _NC * _NS
_CHUNK = 64


def _sc_gather(table, idx, n_rows, out_rows, row0):
    b_per_w = n_rows // _NW
    n_chunk = b_per_w // _CHUNK
    mesh = plsc.VectorSubcoreMesh(core_axis_name="c", subcore_axis_name="s")

    @functools.partial(
        pl.kernel,
        mesh=mesh,
        out_type=jax.ShapeDtypeStruct((out_rows, D_K), jnp.float32),
        scratch_types=[
            pltpu.VMEM((b_per_w,), jnp.int32),
            pltpu.VMEM((_CHUNK, D_K), jnp.float32),
            pltpu.VMEM((_CHUNK, D_K), jnp.float32),
            pltpu.SemaphoreType.DMA,
            pltpu.SemaphoreType.DMA,
            pltpu.SemaphoreType.DMA,
            pltpu.SemaphoreType.DMA,
        ],
    )
    def gather_k(table_hbm, idx_hbm, out_hbm, idx_v, buf0, buf1,
                 gs0, gs1, ws0, ws1):
        wid = lax.axis_index("s") * _NC + lax.axis_index("c")
        base = row0 + wid * b_per_w
        pltpu.sync_copy(idx_hbm.at[pl.ds(wid * b_per_w, b_per_w)], idx_v)
        bufs = (buf0, buf1)
        gsem = (gs0, gs1)
        wsem = (ws0, ws1)

        def start_write(j):
            return pltpu.async_copy(
                bufs[j % 2], out_hbm.at[pl.ds(base + j * _CHUNK, _CHUNK)],
                wsem[j % 2])

        g, w = {}, {}
        for j in range(n_chunk):
            if j >= 2:
                w[j - 2].wait()
            g[j] = pltpu.async_copy(
                table_hbm.at[idx_v.at[pl.ds(j * _CHUNK, _CHUNK)]],
                bufs[j % 2], gsem[j % 2])
            if j >= 1:
                g[j - 1].wait()
                w[j - 1] = start_write(j - 1)
        g[n_chunk - 1].wait()
        w[n_chunk - 1] = start_write(n_chunk - 1)
        if n_chunk >= 2:
            w[n_chunk - 2].wait()
        w[n_chunk - 1].wait()

    return gather_k(table, idx)


_NSPLIT = 4


def kernel(vecs, c_sum, c_count):
    b, r, cdim, k = vecs.shape
    n = b * r * cdim
    v2 = vecs.astype(jnp.float32).reshape(n, k)
    c = jnp.divide(c_sum, jnp.clip(jnp.expand_dims(c_count, -1), 0.01))
    c = c.astype(jnp.float32)
    cn = jnp.einsum('sk->s', jnp.square(c))

    np_ = n // _NSPLIT
    z_parts, lsums = [], []
    hat = None
    for p in range(_NSPLIT):
        z_p, lsum_p = _distances_argmin(v2, c, cn, p, np_)
        if p == 0:
            hat = _sc_gather(c, z_p, np_, n, 0)
        else:
            hat_p = _sc_gather(c, z_p, np_, np_, 0)
            hat = lax.dynamic_update_slice(hat, hat_p, (p * np_, 0))
        z_parts.append(z_p)
        lsums.append(lsum_p[0, 0])

    l_commit = sum(lsums) / (b * r)
    z = jnp.concatenate(z_parts).reshape(b, r, cdim)
    vecs_hat = hat.reshape(b, r, cdim, k).astype(vecs.dtype)
    l_codebook = jnp.zeros((), jnp.float32)
    return vecs_hat, z, l_commit, l_codebook


# speedup vs baseline: 1.2879x; 1.0954x over previous
import functools

import jax
import jax.numpy as jnp
from jax import lax
from jax.experimental import pallas as pl
from jax.experimental.pallas import tpu as pltpu
from jax.experimental.pallas import tpu_sc as plsc

N_CODE = 1024
D_K = 256


_RB = 1024
_NCH = 128
_NCHUNKS = N_CODE // _NCH


def _dist_body(v_ref, c_ref, cn_ref, z_ref, lsum_ref):
    i = pl.program_id(0)

    @pl.when(i == 0)
    def _init():
        lsum_ref[0, 0] = 0.0

    v = v_ref[...]
    c = c_ref[...]
    s = lax.dot_general(v, c, (((1,), (1,)), ((), ())),
                        preferred_element_type=jnp.float32)
    vn = jnp.sum(v * v, axis=1, keepdims=True)
    cn = cn_ref[...]

    def chunk(j):
        lo, hi = j * _NCH, (j + 1) * _NCH
        return (vn + (-2.0) * s[:, lo:hi]) + cn[lo:hi][None, :]

    val = chunk(0)
    jwin = jnp.zeros((_RB, _NCH), jnp.int32)
    for j in range(1, _NCHUNKS):
        d = chunk(j)
        lt = d < val
        val = jnp.minimum(val, d)
        jwin = jnp.where(lt, j, jwin)
    g = jwin * _NCH + lax.broadcasted_iota(jnp.int32, (_RB, _NCH), 1)
    valT = val.T
    gT = g.T
    m = jnp.min(valT, axis=0, keepdims=True)
    z = jnp.min(jnp.where(valT == m, gT, N_CODE), axis=0)
    z_ref[...] = z.astype(jnp.int32)
    lsum_ref[0, 0] += jnp.sum(jnp.maximum(m, 0.0))


def _distances_argmin(v2, c, cn, part, n_part):
    steps = n_part // _RB
    grid = (steps,)
    z, lsum = pl.pallas_call(
        _dist_body,
        grid=grid,
        in_specs=[
            pl.BlockSpec((_RB, D_K), lambda i, p=part, s=steps: (p * s + i, 0)),
            pl.BlockSpec((N_CODE, D_K), lambda i: (0, 0)),
            pl.BlockSpec((N_CODE,), lambda i: (0,)),
        ],
        out_specs=[
            pl.BlockSpec((_RB,), lambda i: (i,)),
            pl.BlockSpec(memory_space=pltpu.SMEM),
        ],
        out_shape=[
            jax.ShapeDtypeStruct((n_part,), jnp.int32),
            jax.ShapeDtypeStruct((1, 1), jnp.float32),
        ],
    )(v2, c, cn)
    return z, lsum


_NC = 2
_NS = 16
_NW = _NC * _NS
_CHUNK = 64


def _sc_gather(table, idx, n_rows, out_rows, row0):
    b_per_w = n_rows // _NW
    n_chunk = b_per_w // _CHUNK
    mesh = plsc.VectorSubcoreMesh(core_axis_name="c", subcore_axis_name="s")

    @functools.partial(
        pl.kernel,
        mesh=mesh,
        out_type=jax.ShapeDtypeStruct((out_rows, D_K), jnp.float32),
        scratch_types=[
            pltpu.VMEM((b_per_w,), jnp.int32),
            pltpu.VMEM((_CHUNK, D_K), jnp.float32),
            pltpu.VMEM((_CHUNK, D_K), jnp.float32),
            pltpu.SemaphoreType.DMA,
            pltpu.SemaphoreType.DMA,
            pltpu.SemaphoreType.DMA,
            pltpu.SemaphoreType.DMA,
        ],
    )
    def gather_k(table_hbm, idx_hbm, out_hbm, idx_v, buf0, buf1,
                 gs0, gs1, ws0, ws1):
        wid = lax.axis_index("s") * _NC + lax.axis_index("c")
        base = row0 + wid * b_per_w
        pltpu.sync_copy(idx_hbm.at[pl.ds(wid * b_per_w, b_per_w)], idx_v)
        bufs = (buf0, buf1)
        gsem = (gs0, gs1)
        wsem = (ws0, ws1)

        def start_write(j):
            return pltpu.async_copy(
                bufs[j % 2], out_hbm.at[pl.ds(base + j * _CHUNK, _CHUNK)],
                wsem[j % 2])

        g, w = {}, {}
        for j in range(n_chunk):
            if j >= 2:
                w[j - 2].wait()
            g[j] = pltpu.async_copy(
                table_hbm.at[idx_v.at[pl.ds(j * _CHUNK, _CHUNK)]],
                bufs[j % 2], gsem[j % 2])
            if j >= 1:
                g[j - 1].wait()
                w[j - 1] = start_write(j - 1)
        g[n_chunk - 1].wait()
        w[n_chunk - 1] = start_write(n_chunk - 1)
        if n_chunk >= 2:
            w[n_chunk - 2].wait()
        w[n_chunk - 1].wait()

    return gather_k(table, idx)


_NSPLIT = 4


def kernel(vecs, c_sum, c_count):
    b, r, cdim, k = vecs.shape
    n = b * r * cdim
    v2 = vecs.astype(jnp.float32).reshape(n, k)
    c = jnp.divide(c_sum, jnp.clip(jnp.expand_dims(c_count, -1), 0.01))
    c = c.astype(jnp.float32)
    cn = jnp.einsum('sk->s', jnp.square(c))

    np_ = n // _NSPLIT
    z_parts, lsums = [], []
    hat = None
    for p in range(_NSPLIT):
        z_p, lsum_p = _distances_argmin(v2, c, cn, p, np_)
        if p == 0:
            hat = _sc_gather(c, z_p, np_, n, 0)
        else:
            hat_p = _sc_gather(c, z_p, np_, np_, 0)
            hat = lax.dynamic_update_slice(hat, hat_p, (p * np_, 0))
        z_parts.append(z_p)
        lsums.append(lsum_p[0, 0])

    l_commit = sum(lsums) / (b * r)
    z = jnp.concatenate(z_parts).reshape(b, r, cdim)
    vecs_hat = hat.reshape(b, r, cdim, k).astype(vecs.dtype)
    l_codebook = jnp.zeros((), jnp.float32)
    return vecs_hat, z, l_commit, l_codebook
